# Initial kernel scaffold; baseline (speedup 1.0000x reference)
import jax
import jax.numpy as jnp
from jax.experimental import pallas as pl


def kernel(x, edge_index, Wpre0, bpre0, Wpost0, bpost0, Wlin0, blin0, Wpre1, bpre1, Wpost1, bpost1, Wlin1, blin1, Wout, bout):
    raise NotImplementedError("write your pallas kernel here")


# speedup vs baseline: 1.1464x; 1.1464x over previous
import functools
import math

import jax
import jax.numpy as jnp
from jax.experimental import pallas as pl
from jax.experimental.pallas import tpu as pltpu

N_NODES = 10000
DIM = 256
AVG_LOG = math.log(17.0)
PAD_N = 10240
BLK = 512


def _post_body(x_ref, av_ref, sb_ref, sb2_ref, mn_ref, mx_ref, deg_ref,
               wpx_ref, wpid_ref, wpamp_ref, wpatt_ref, bpost_ref,
               wlin_ref, blin_ref, out_ref, *, relu):
    d = deg_ref[:, 0:1]
    cnt = jnp.maximum(d, 1.0)
    has = d > 0.0
    av = av_ref[:]
    sb = sb_ref[:]
    s = d * av + sb
    mean = s / cnt
    msq = (d * av * av + 2.0 * av * sb + sb2_ref[:]) / cnt
    std = jnp.sqrt(jax.nn.relu(msq - mean * mean) + 1e-5)
    mn = jnp.where(has, av + mn_ref[:], 0.0)
    mx = jnp.where(has, av + mx_ref[:], 0.0)
    agg = jnp.concatenate([mean, s, std, mn, mx], axis=1)
    dl = jnp.log(cnt + 1.0)
    a_s = dl / AVG_LOG
    t_s = AVG_LOG / dl
    dot = functools.partial(jnp.dot, preferred_element_type=jnp.float32)
    out = (dot(x_ref[:], wpx_ref[:]) + dot(agg, wpid_ref[:])
           + dot(agg * a_s, wpamp_ref[:]) + dot(agg * t_s, wpatt_ref[:])
           + bpost_ref[:])
    out = dot(out, wlin_ref[:]) + blin_ref[:]
    if relu:
        out = jax.nn.relu(out)
    out_ref[:] = out


def _pna_post(x, av, sb, sb2, mn, mx, deg, Wpost, bpost, Wlin, blin, relu):
    wpx = Wpost[0:DIM]
    wpid = Wpost[DIM:DIM + 5 * DIM]
    wpamp = Wpost[DIM + 5 * DIM:DIM + 10 * DIM]
    wpatt = Wpost[DIM + 10 * DIM:DIM + 15 * DIM]
    n_blk = PAD_N // BLK
    row = lambda i: (i, 0)
    full = lambda i: (0, 0)
    grid_spec = pl.GridSpec(
        grid=(n_blk,),
        in_specs=[
            pl.BlockSpec((BLK, DIM), row),
            pl.BlockSpec((BLK, DIM), row),
            pl.BlockSpec((BLK, DIM), row),
            pl.BlockSpec((BLK, DIM), row),
            pl.BlockSpec((BLK, DIM), row),
            pl.BlockSpec((BLK, DIM), row),
            pl.BlockSpec((BLK, 1), row),
            pl.BlockSpec((DIM, DIM), full),
            pl.BlockSpec((5 * DIM, DIM), full),
            pl.BlockSpec((5 * DIM, DIM), full),
            pl.BlockSpec((5 * DIM, DIM), full),
            pl.BlockSpec((1, DIM), full),
            pl.BlockSpec((DIM, DIM), full),
            pl.BlockSpec((1, DIM), full),
        ],
        out_specs=pl.BlockSpec((BLK, DIM), row),
    )
    return pl.pallas_call(
        functools.partial(_post_body, relu=relu),
        grid_spec=grid_spec,
        out_shape=jax.ShapeDtypeStruct((PAD_N, DIM), jnp.float32),
    )(x, av, sb, sb2, mn, mx, deg, wpx, wpid, wpamp, wpatt,
      bpost[None, :], Wlin, blin[None, :])


def _layer(x, src, dst, deg, Wpre, bpre, Wpost, bpost, Wlin, blin, relu):
    a = x @ Wpre[:DIM] + bpre
    b = x @ Wpre[DIM:]
    bs = b[src]
    sb = jax.ops.segment_sum(bs, dst, num_segments=N_NODES)
    sb2 = jax.ops.segment_sum(bs * bs, dst, num_segments=N_NODES)
    mnb = jax.ops.segment_min(bs, dst, num_segments=N_NODES)
    mxb = jax.ops.segment_max(bs, dst, num_segments=N_NODES)
    has = (deg > 0)[:, None]
    mnb = jnp.where(has, mnb, 0.0)
    mxb = jnp.where(has, mxb, 0.0)
    pad = lambda t: jnp.pad(t, ((0, PAD_N - N_NODES), (0, 0)))
    out = _pna_post(pad(x), pad(a), pad(sb), pad(sb2), pad(mnb), pad(mxb),
                    pad(deg[:, None]), Wpost, bpost, Wlin, blin, relu)
    return out[:N_NODES]


def kernel(x, edge_index, Wpre0, bpre0, Wpost0, bpost0, Wlin0, blin0,
           Wpre1, bpre1, Wpost1, bpost1, Wlin1, blin1, Wout, bout):
    src = edge_index[0]
    dst = edge_index[1]
    deg = jax.ops.segment_sum(jnp.ones_like(dst, dtype=jnp.float32), dst,
                              num_segments=N_NODES)
    h = _layer(x, src, dst, deg, Wpre0, bpre0, Wpost0, bpost0, Wlin0, blin0,
               relu=True)
    h = _layer(h, src, dst, deg, Wpre1, bpre1, Wpost1, bpost1, Wlin1, blin1,
               relu=True)
    return jnp.squeeze(h @ Wout + bout)


# speedup vs baseline: 3.0084x; 2.6244x over previous
import functools
import math

import jax
import jax.numpy as jnp
from jax import lax
from jax.experimental import pallas as pl
from jax.experimental.pallas import tpu as pltpu
from jax.experimental.pallas import tpu_sc as plsc

N_NODES = 10000
N_EDGES = 160000
DIM = 256
HALF = 128
AVG_LOG = math.log(17.0)
PAD_N = 10240
BLK = 512
NPART = 16
CHUNK = 128
SP = N_EDGES + 2 * CHUNK
OFF_PAD = 10008
INF = float("inf")


def _ext(buf, j):
    base = (j // 16) * 16
    w = buf[pl.ds(base, 16)]
    lane = j - base
    sel = jnp.where(lax.iota(jnp.int32, 16) == lane,
                    w.astype(jnp.float32), 0.0)
    return jnp.sum(sel).astype(jnp.int32)


def _acc_init(accvm):
    zero = jnp.zeros((16,), jnp.float32)
    pinf = jnp.full((16,), INF, jnp.float32)
    ninf = jnp.full((16,), -INF, jnp.float32)
    for k in range(8):
        accvm[pl.ds(k * 16, 16)] = zero
        accvm[pl.ds(HALF + k * 16, 16)] = zero
        accvm[pl.ds(2 * HALF + k * 16, 16)] = pinf
        accvm[pl.ds(3 * HALF + k * 16, 16)] = ninf


def _sc_body(b2_hbm, srcs_hbm, off_hbm, vb_hbm, out_hbm,
             offv, vbv, idxA, idxB, rowA, rowB, accvm, semA, semB):
    nc = 2
    wid = lax.axis_index("s") * nc + lax.axis_index("c")
    p = wid % NPART
    h = wid // NPART

    pltpu.sync_copy(off_hbm, offv)
    pltpu.sync_copy(vb_hbm, vbv)

    v0 = _ext(vbv, p)
    v1 = _ext(vbv, p + 1)
    e_lo = _ext(offv, v0)
    e_hi = _ext(offv, v1)
    cb0 = (e_lo // 8) * 8
    n_chunks = (e_hi - cb0 + CHUNK - 1) // CHUNK

    def walk(v_from, e_cur):
        def cond(vv):
            return jnp.logical_and(vv < v1, _ext(offv, vv + 1) <= e_cur)
        return lax.while_loop(cond, lambda vv: vv + 1, v_from)

    v_init = walk(v0, e_lo)
    e1_init = jnp.where(v_init < v1, _ext(offv, v_init + 1), e_hi)
    _acc_init(accvm)

    def stage_and_start(m, idxv, rowv, sem):
        cb = cb0 + m * CHUNK
        pltpu.sync_copy(srcs_hbm.at[pl.ds(cb, CHUNK)], idxv)
        for t in range(CHUNK // 16):
            sl = pl.ds(t * 16, 16)
            idxv[sl] = idxv[sl] * 2 + h
        pltpu.make_async_copy(b2_hbm.at[idxv], rowv, sem).start()

    @pl.when(n_chunks > 0)
    def _prime():
        stage_and_start(0, idxA, rowA, semA)

    def process(m, carry, rowv):
        cb = cb0 + m * CHUNK
        ce = jnp.minimum(cb + CHUNK, e_hi)

        def run(carry):
            e, v, e1 = carry
            run_end = jnp.minimum(e1, ce)
            rows = run_end - e
            rb = e - cb
            for k in range(8):
                sl = pl.ds(k * 16, 16)
                s0 = accvm[pl.ds(k * 16, 16)]
                q0 = accvm[pl.ds(HALF + k * 16, 16)]
                mn0 = accvm[pl.ds(2 * HALF + k * 16, 16)]
                mx0 = accvm[pl.ds(3 * HALF + k * 16, 16)]

                def rbody(r, c4, _sl=sl):
                    s, q, mn, mx = c4
                    xv = rowv[rb + r, _sl]
                    return (s + xv, q + xv * xv,
                            jnp.minimum(mn, xv), jnp.maximum(mx, xv))

                s0, q0, mn0, mx0 = lax.fori_loop(0, rows, rbody,
                                                 (s0, q0, mn0, mx0))
                accvm[pl.ds(k * 16, 16)] = s0
                accvm[pl.ds(HALF + k * 16, 16)] = q0
                accvm[pl.ds(2 * HALF + k * 16, 16)] = mn0
                accvm[pl.ds(3 * HALF + k * 16, 16)] = mx0

            done = run_end == e1

            @pl.when(done)
            def _flush():
                pltpu.sync_copy(accvm, out_hbm.at[h, v])
                _acc_init(accvm)

            v_new = jnp.where(done, walk(v + 1, run_end), v)
            e1_new = jnp.where(done,
                               jnp.where(v_new < v1,
                                         _ext(offv, v_new + 1), e_hi),
                               e1)
            return (run_end, v_new, e1_new)

        return lax.while_loop(lambda c: c[0] < ce, run, carry)

    def chunk_pair(m2, carry):
        for b in range(2):
            m = m2 * 2 + b
            idx_n, row_n, sem_n = (idxB, rowB, semB) if b == 0 else (idxA, rowA, semA)
            idx_c, row_c, sem_c = (idxA, rowA, semA) if b == 0 else (idxB, rowB, semB)

            @pl.when(m + 1 < n_chunks)
            def _start():
                stage_and_start(m + 1, idx_n, row_n, sem_n)

            @pl.when(m < n_chunks)
            def _wait():
                pltpu.make_async_copy(b2_hbm.at[idx_c], row_c, sem_c).wait()

            carry = process(m, carry, row_c)
        return carry

    n2 = (n_chunks + 1) // 2
    lax.fori_loop(0, n2, chunk_pair, (e_lo, v_init, e1_init))


def _sc_segment_reduce(b2, src_s, off, vb):
    mesh = plsc.VectorSubcoreMesh(core_axis_name="c", subcore_axis_name="s",
                                  num_cores=2, num_subcores=16)
    return pl.kernel(
        _sc_body,
        out_type=jax.ShapeDtypeStruct((2, N_NODES, 4 * HALF), jnp.float32),
        mesh=mesh,
        compiler_params=pltpu.CompilerParams(needs_layout_passes=False),
        scratch_types=[
            pltpu.VMEM((OFF_PAD,), jnp.int32),
            pltpu.VMEM((24,), jnp.int32),
            pltpu.VMEM((CHUNK,), jnp.int32),
            pltpu.VMEM((CHUNK,), jnp.int32),
            pltpu.VMEM((CHUNK, HALF), jnp.float32),
            pltpu.VMEM((CHUNK, HALF), jnp.float32),
            pltpu.VMEM((4 * HALF,), jnp.float32),
            pltpu.SemaphoreType.DMA,
            pltpu.SemaphoreType.DMA,
        ],
    )(b2, src_s, off, vb)


def _post_body(x_ref, av_ref, h0_ref, h1_ref, deg_ref,
               wpx_ref, wpid_ref, wpamp_ref, wpatt_ref, bpost_ref,
               wlin_ref, blin_ref, out_ref, *, relu):
    d = deg_ref[:, 0:1]
    cnt = jnp.maximum(d, 1.0)
    has = d > 0.0
    av = av_ref[:]
    h0 = h0_ref[:]
    h1 = h1_ref[:]
    cat = lambda a: jnp.concatenate(
        [h0[:, a * HALF:(a + 1) * HALF], h1[:, a * HALF:(a + 1) * HALF]], axis=1)
    sb = jnp.where(has, cat(0), 0.0)
    sq = jnp.where(has, cat(1), 0.0)
    s = d * av + sb
    mean = s / cnt
    msq = (d * av * av + 2.0 * av * sb + sq) / cnt
    std = jnp.sqrt(jax.nn.relu(msq - mean * mean) + 1e-5)
    mn = jnp.where(has, av + cat(2), 0.0)
    mx = jnp.where(has, av + cat(3), 0.0)
    agg = jnp.concatenate([mean, s, std, mn, mx], axis=1)
    dl = jnp.log(cnt + 1.0)
    a_s = dl / AVG_LOG
    t_s = AVG_LOG / dl
    dot = functools.partial(jnp.dot, preferred_element_type=jnp.float32)
    out = (dot(x_ref[:], wpx_ref[:]) + dot(agg, wpid_ref[:])
           + dot(agg * a_s, wpamp_ref[:]) + dot(agg * t_s, wpatt_ref[:])
           + bpost_ref[:])
    out = dot(out, wlin_ref[:]) + blin_ref[:]
    if relu:
        out = jax.nn.relu(out)
    out_ref[:] = out


def _pna_post(x, av, h0, h1, deg, Wpost, bpost, Wlin, blin, relu):
    wpx = Wpost[0:DIM]
    wpid = Wpost[DIM:DIM + 5 * DIM]
    wpamp = Wpost[DIM + 5 * DIM:DIM + 10 * DIM]
    wpatt = Wpost[DIM + 10 * DIM:DIM + 15 * DIM]
    n_blk = PAD_N // BLK
    row = lambda i: (i, 0)
    full = lambda i: (0, 0)
    grid_spec = pl.GridSpec(
        grid=(n_blk,),
        in_specs=[
            pl.BlockSpec((BLK, DIM), row),
            pl.BlockSpec((BLK, DIM), row),
            pl.BlockSpec((BLK, 4 * HALF), row),
            pl.BlockSpec((BLK, 4 * HALF), row),
            pl.BlockSpec((BLK, 1), row),
            pl.BlockSpec((DIM, DIM), full),
            pl.BlockSpec((5 * DIM, DIM), full),
            pl.BlockSpec((5 * DIM, DIM), full),
            pl.BlockSpec((5 * DIM, DIM), full),
            pl.BlockSpec((1, DIM), full),
            pl.BlockSpec((DIM, DIM), full),
            pl.BlockSpec((1, DIM), full),
        ],
        out_specs=pl.BlockSpec((BLK, DIM), row),
    )
    return pl.pallas_call(
        functools.partial(_post_body, relu=relu),
        grid_spec=grid_spec,
        out_shape=jax.ShapeDtypeStruct((PAD_N, DIM), jnp.float32),
    )(x, av, h0, h1, deg, wpx, wpid, wpamp, wpatt,
      bpost[None, :], Wlin, blin[None, :])


def _layer(x, src_s, off, vb, deg, Wpre, bpre, Wpost, bpost, Wlin, blin, relu):
    a = x @ Wpre[:DIM] + bpre
    b = x @ Wpre[DIM:]
    b2 = b.reshape(2 * N_NODES, HALF)
    rec = _sc_segment_reduce(b2, src_s, off, vb)
    pad = lambda t: jnp.pad(t, ((0, PAD_N - N_NODES), (0, 0)))
    out = _pna_post(pad(x), pad(a), pad(rec[0]), pad(rec[1]),
                    pad(deg[:, None]), Wpost, bpost, Wlin, blin, relu)
    return out[:N_NODES]


def kernel(x, edge_index, Wpre0, bpre0, Wpost0, bpost0, Wlin0, blin0,
           Wpre1, bpre1, Wpost1, bpost1, Wlin1, blin1, Wout, bout):
    src = edge_index[0]
    dst = edge_index[1]
    order = jnp.argsort(dst)
    src_s = jnp.take(src, order).astype(jnp.int32)
    dst_s = jnp.take(dst, order).astype(jnp.int32)
    off = jnp.searchsorted(dst_s, jnp.arange(N_NODES + 1, dtype=jnp.int32),
                           side="left").astype(jnp.int32)
    deg = (off[1:] - off[:-1]).astype(jnp.float32)
    targets = (jnp.arange(NPART + 1, dtype=jnp.int32) * (N_EDGES // NPART))
    vb = jnp.searchsorted(off, targets, side="left").astype(jnp.int32)
    vb = vb.at[0].set(0).at[NPART].set(N_NODES)
    src_sp = jnp.pad(src_s, (0, SP - N_EDGES))
    off_p = jnp.pad(off, (0, OFF_PAD - (N_NODES + 1)),
                    constant_values=N_EDGES)
    vb_p = jnp.pad(vb, (0, 24 - (NPART + 1)))

    h = _layer(x, src_sp, off_p, vb_p, deg, Wpre0, bpre0, Wpost0, bpost0,
               Wlin0, blin0, relu=True)
    h = _layer(h, src_sp, off_p, vb_p, deg, Wpre1, bpre1, Wpost1, bpost1,
               Wlin1, blin1, relu=True)
    return jnp.squeeze(h @ Wout + bout)


# speedup vs baseline: 3.8810x; 1.2900x over previous
import functools
import math

import jax
import jax.numpy as jnp
from jax import lax
from jax.experimental import pallas as pl
from jax.experimental.pallas import tpu as pltpu
from jax.experimental.pallas import tpu_sc as plsc

N_NODES = 10000
N_EDGES = 160000
DIM = 256
HALF = 128
AVG_LOG = math.log(17.0)
PAD_N = 10240
BLK = 512
NPART = 16
CHUNK = 128
SP = N_EDGES + 2 * CHUNK
OFF_PAD = 10008
INF = float("inf")


def _ext(buf, j):
    base = (j // 16) * 16
    w = buf[pl.ds(base, 16)]
    lane = j - base
    sel = jnp.where(lax.iota(jnp.int32, 16) == lane,
                    w.astype(jnp.float32), 0.0)
    return jnp.sum(sel).astype(jnp.int32)


def _acc_init(accvm):
    zero = jnp.zeros((16,), jnp.float32)
    pinf = jnp.full((16,), INF, jnp.float32)
    ninf = jnp.full((16,), -INF, jnp.float32)
    for k in range(8):
        accvm[pl.ds(k * 16, 16)] = zero
        accvm[pl.ds(HALF + k * 16, 16)] = zero
        accvm[pl.ds(2 * HALF + k * 16, 16)] = pinf
        accvm[pl.ds(3 * HALF + k * 16, 16)] = ninf


def _sc_body(b2_hbm, srcs_hbm, off_hbm, vb_hbm, out_hbm,
             offv, vbv, idxA, idxB, rowA, rowB, accvm, semA, semB):
    nc = 2
    wid = lax.axis_index("s") * nc + lax.axis_index("c")
    p = wid % NPART
    h = wid // NPART

    pltpu.sync_copy(off_hbm, offv)
    pltpu.sync_copy(vb_hbm, vbv)

    v0 = _ext(vbv, p)
    v1 = _ext(vbv, p + 1)
    e_lo = _ext(offv, v0)
    e_hi = _ext(offv, v1)
    cb0 = (e_lo // 8) * 8
    n_chunks = (e_hi - cb0 + CHUNK - 1) // CHUNK

    def walk(v_from, e_cur):
        def cond(vv):
            return jnp.logical_and(vv < v1, _ext(offv, vv + 1) <= e_cur)
        return lax.while_loop(cond, lambda vv: vv + 1, v_from)

    v_init = walk(v0, e_lo)
    e1_init = jnp.where(v_init < v1, _ext(offv, v_init + 1), e_hi)
    _acc_init(accvm)

    def stage_and_start(m, idxv, rowv, sem):
        cb = cb0 + m * CHUNK
        pltpu.sync_copy(srcs_hbm.at[pl.ds(cb, CHUNK)], idxv)
        for t in range(CHUNK // 16):
            sl = pl.ds(t * 16, 16)
            idxv[sl] = idxv[sl] * 2 + h
        pltpu.make_async_copy(b2_hbm.at[idxv], rowv, sem).start()

    @pl.when(n_chunks > 0)
    def _prime():
        stage_and_start(0, idxA, rowA, semA)

    def process(m, carry, rowv):
        cb = cb0 + m * CHUNK
        ce = jnp.minimum(cb + CHUNK, e_hi)

        def run(carry):
            e, v, e1 = carry
            run_end = jnp.minimum(e1, ce)
            rows = run_end - e
            rb = e - cb
            accs = []
            for k in range(8):
                accs += [accvm[pl.ds(k * 16, 16)],
                         accvm[pl.ds(HALF + k * 16, 16)],
                         accvm[pl.ds(2 * HALF + k * 16, 16)],
                         accvm[pl.ds(3 * HALF + k * 16, 16)]]

            def rbody(r, acc):
                out = []
                for k in range(8):
                    xv = rowv[rb + r, pl.ds(k * 16, 16)]
                    s, q, mn, mx = acc[4 * k:4 * k + 4]
                    out += [s + xv, q + xv * xv,
                            jnp.minimum(mn, xv), jnp.maximum(mx, xv)]
                return tuple(out)

            accs = lax.fori_loop(0, rows, rbody, tuple(accs))
            for k in range(8):
                accvm[pl.ds(k * 16, 16)] = accs[4 * k]
                accvm[pl.ds(HALF + k * 16, 16)] = accs[4 * k + 1]
                accvm[pl.ds(2 * HALF + k * 16, 16)] = accs[4 * k + 2]
                accvm[pl.ds(3 * HALF + k * 16, 16)] = accs[4 * k + 3]

            done = run_end == e1

            @pl.when(done)
            def _flush():
                pltpu.sync_copy(accvm, out_hbm.at[h, v])
                _acc_init(accvm)

            v_new = jnp.where(done, walk(v + 1, run_end), v)
            e1_new = jnp.where(done,
                               jnp.where(v_new < v1,
                                         _ext(offv, v_new + 1), e_hi),
                               e1)
            return (run_end, v_new, e1_new)

        return lax.while_loop(lambda c: c[0] < ce, run, carry)

    def chunk_pair(m2, carry):
        for b in range(2):
            m = m2 * 2 + b
            idx_n, row_n, sem_n = (idxB, rowB, semB) if b == 0 else (idxA, rowA, semA)
            idx_c, row_c, sem_c = (idxA, rowA, semA) if b == 0 else (idxB, rowB, semB)

            @pl.when(m + 1 < n_chunks)
            def _start():
                stage_and_start(m + 1, idx_n, row_n, sem_n)

            @pl.when(m < n_chunks)
            def _wait():
                pltpu.make_async_copy(b2_hbm.at[idx_c], row_c, sem_c).wait()

            carry = process(m, carry, row_c)
        return carry

    n2 = (n_chunks + 1) // 2
    lax.fori_loop(0, n2, chunk_pair, (e_lo, v_init, e1_init))


def _sc_segment_reduce(b2, src_s, off, vb):
    mesh = plsc.VectorSubcoreMesh(core_axis_name="c", subcore_axis_name="s",
                                  num_cores=2, num_subcores=16)
    return pl.kernel(
        _sc_body,
        out_type=jax.ShapeDtypeStruct((2, N_NODES, 4 * HALF), jnp.float32),
        mesh=mesh,
        compiler_params=pltpu.CompilerParams(needs_layout_passes=False),
        scratch_types=[
            pltpu.VMEM((OFF_PAD,), jnp.int32),
            pltpu.VMEM((24,), jnp.int32),
            pltpu.VMEM((CHUNK,), jnp.int32),
            pltpu.VMEM((CHUNK,), jnp.int32),
            pltpu.VMEM((CHUNK, HALF), jnp.float32),
            pltpu.VMEM((CHUNK, HALF), jnp.float32),
            pltpu.VMEM((4 * HALF,), jnp.float32),
            pltpu.SemaphoreType.DMA,
            pltpu.SemaphoreType.DMA,
        ],
    )(b2, src_s, off, vb)


def _post_body(x_ref, av_ref, h0_ref, h1_ref, deg_ref,
               wpx_ref, wpid_ref, wpamp_ref, wpatt_ref, bpost_ref,
               wlin_ref, blin_ref, out_ref, *, relu):
    d = deg_ref[:, 0:1]
    cnt = jnp.maximum(d, 1.0)
    has = d > 0.0
    av = av_ref[:]
    h0 = h0_ref[:]
    h1 = h1_ref[:]
    cat = lambda a: jnp.concatenate(
        [h0[:, a * HALF:(a + 1) * HALF], h1[:, a * HALF:(a + 1) * HALF]], axis=1)
    sb = jnp.where(has, cat(0), 0.0)
    sq = jnp.where(has, cat(1), 0.0)
    s = d * av + sb
    mean = s / cnt
    msq = (d * av * av + 2.0 * av * sb + sq) / cnt
    std = jnp.sqrt(jax.nn.relu(msq - mean * mean) + 1e-5)
    mn = jnp.where(has, av + cat(2), 0.0)
    mx = jnp.where(has, av + cat(3), 0.0)
    agg = jnp.concatenate([mean, s, std, mn, mx], axis=1)
    dl = jnp.log(cnt + 1.0)
    a_s = dl / AVG_LOG
    t_s = AVG_LOG / dl
    dot = functools.partial(jnp.dot, preferred_element_type=jnp.float32)
    out = (dot(x_ref[:], wpx_ref[:]) + dot(agg, wpid_ref[:])
           + dot(agg * a_s, wpamp_ref[:]) + dot(agg * t_s, wpatt_ref[:])
           + bpost_ref[:])
    out = dot(out, wlin_ref[:]) + blin_ref[:]
    if relu:
        out = jax.nn.relu(out)
    out_ref[:] = out


def _pna_post(x, av, h0, h1, deg, Wpost, bpost, Wlin, blin, relu):
    wpx = Wpost[0:DIM]
    wpid = Wpost[DIM:DIM + 5 * DIM]
    wpamp = Wpost[DIM + 5 * DIM:DIM + 10 * DIM]
    wpatt = Wpost[DIM + 10 * DIM:DIM + 15 * DIM]
    n_blk = PAD_N // BLK
    row = lambda i: (i, 0)
    full = lambda i: (0, 0)
    grid_spec = pl.GridSpec(
        grid=(n_blk,),
        in_specs=[
            pl.BlockSpec((BLK, DIM), row),
            pl.BlockSpec((BLK, DIM), row),
            pl.BlockSpec((BLK, 4 * HALF), row),
            pl.BlockSpec((BLK, 4 * HALF), row),
            pl.BlockSpec((BLK, 1), row),
            pl.BlockSpec((DIM, DIM), full),
            pl.BlockSpec((5 * DIM, DIM), full),
            pl.BlockSpec((5 * DIM, DIM), full),
            pl.BlockSpec((5 * DIM, DIM), full),
            pl.BlockSpec((1, DIM), full),
            pl.BlockSpec((DIM, DIM), full),
            pl.BlockSpec((1, DIM), full),
        ],
        out_specs=pl.BlockSpec((BLK, DIM), row),
    )
    return pl.pallas_call(
        functools.partial(_post_body, relu=relu),
        grid_spec=grid_spec,
        out_shape=jax.ShapeDtypeStruct((PAD_N, DIM), jnp.float32),
    )(x, av, h0, h1, deg, wpx, wpid, wpamp, wpatt,
      bpost[None, :], Wlin, blin[None, :])


def _layer(x, src_s, off, vb, deg, Wpre, bpre, Wpost, bpost, Wlin, blin, relu):
    a = x @ Wpre[:DIM] + bpre
    b = x @ Wpre[DIM:]
    b2 = b.reshape(2 * N_NODES, HALF)
    rec = _sc_segment_reduce(b2, src_s, off, vb)
    pad = lambda t: jnp.pad(t, ((0, PAD_N - N_NODES), (0, 0)))
    out = _pna_post(pad(x), pad(a), pad(rec[0]), pad(rec[1]),
                    pad(deg[:, None]), Wpost, bpost, Wlin, blin, relu)
    return out[:N_NODES]


def kernel(x, edge_index, Wpre0, bpre0, Wpost0, bpost0, Wlin0, blin0,
           Wpre1, bpre1, Wpost1, bpost1, Wlin1, blin1, Wout, bout):
    src = edge_index[0]
    dst = edge_index[1]
    order = jnp.argsort(dst)
    src_s = jnp.take(src, order).astype(jnp.int32)
    dst_s = jnp.take(dst, order).astype(jnp.int32)
    off = jnp.searchsorted(dst_s, jnp.arange(N_NODES + 1, dtype=jnp.int32),
                           side="left").astype(jnp.int32)
    deg = (off[1:] - off[:-1]).astype(jnp.float32)
    targets = (jnp.arange(NPART + 1, dtype=jnp.int32) * (N_EDGES // NPART))
    vb = jnp.searchsorted(off, targets, side="left").astype(jnp.int32)
    vb = vb.at[0].set(0).at[NPART].set(N_NODES)
    src_sp = jnp.pad(src_s, (0, SP - N_EDGES))
    off_p = jnp.pad(off, (0, OFF_PAD - (N_NODES + 1)),
                    constant_values=N_EDGES)
    vb_p = jnp.pad(vb, (0, 24 - (NPART + 1)))

    h = _layer(x, src_sp, off_p, vb_p, deg, Wpre0, bpre0, Wpost0, bpost0,
               Wlin0, blin0, relu=True)
    h = _layer(h, src_sp, off_p, vb_p, deg, Wpre1, bpre1, Wpost1, bpost1,
               Wlin1, blin1, relu=True)
    return jnp.squeeze(h @ Wout + bout)


# speedup vs baseline: 4.0719x; 1.0492x over previous
import functools
import math

import jax
import jax.numpy as jnp
from jax import lax
from jax.experimental import pallas as pl
from jax.experimental.pallas import tpu as pltpu
from jax.experimental.pallas import tpu_sc as plsc

N_NODES = 10000
N_EDGES = 160000
DIM = 256
HALF = 128
AVG_LOG = math.log(17.0)
BLK = 400
NPART = 16
CHUNK = 128
SP = N_EDGES + 2 * CHUNK
OFF_PAD = 10008
INF = float("inf")


def _ext(buf, j):
    base = (j // 16) * 16
    w = buf[pl.ds(base, 16)]
    lane = j - base
    sel = jnp.where(lax.iota(jnp.int32, 16) == lane,
                    w.astype(jnp.float32), 0.0)
    return jnp.sum(sel).astype(jnp.int32)


def _acc_init(accvm):
    zero = jnp.zeros((16,), jnp.float32)
    pinf = jnp.full((16,), INF, jnp.float32)
    ninf = jnp.full((16,), -INF, jnp.float32)
    for k in range(8):
        accvm[pl.ds(k * 16, 16)] = zero
        accvm[pl.ds(HALF + k * 16, 16)] = zero
        accvm[pl.ds(2 * HALF + k * 16, 16)] = pinf
        accvm[pl.ds(3 * HALF + k * 16, 16)] = ninf


def _sc_body(b2_hbm, srcs_hbm, off_hbm, vb_hbm, out_hbm,
             offv, vbv, idxA, idxB, rowA, rowB, accvm, semA, semB):
    nc = 2
    wid = lax.axis_index("s") * nc + lax.axis_index("c")
    p = wid % NPART
    h = wid // NPART

    pltpu.sync_copy(off_hbm, offv)
    pltpu.sync_copy(vb_hbm, vbv)

    v0 = _ext(vbv, p)
    v1 = _ext(vbv, p + 1)
    e_lo = _ext(offv, v0)
    e_hi = _ext(offv, v1)
    cb0 = (e_lo // 8) * 8
    n_chunks = (e_hi - cb0 + CHUNK - 1) // CHUNK

    def walk(v_from, e_cur):
        def cond(vv):
            return jnp.logical_and(vv < v1, _ext(offv, vv + 1) <= e_cur)
        return lax.while_loop(cond, lambda vv: vv + 1, v_from)

    v_init = walk(v0, e_lo)
    e1_init = jnp.where(v_init < v1, _ext(offv, v_init + 1), e_hi)
    _acc_init(accvm)

    def stage_and_start(m, idxv, rowv, sem):
        cb = cb0 + m * CHUNK
        pltpu.sync_copy(srcs_hbm.at[pl.ds(cb, CHUNK)], idxv)
        for t in range(CHUNK // 16):
            sl = pl.ds(t * 16, 16)
            idxv[sl] = idxv[sl] * 2 + h
        pltpu.make_async_copy(b2_hbm.at[idxv], rowv, sem).start()

    @pl.when(n_chunks > 0)
    def _prime():
        stage_and_start(0, idxA, rowA, semA)

    def process(m, carry, rowv):
        cb = cb0 + m * CHUNK
        ce = jnp.minimum(cb + CHUNK, e_hi)

        def run(carry):
            e, v, e1 = carry
            run_end = jnp.minimum(e1, ce)
            rows = run_end - e
            rb = e - cb
            accs = []
            for k in range(8):
                accs += [accvm[pl.ds(k * 16, 16)],
                         accvm[pl.ds(HALF + k * 16, 16)],
                         accvm[pl.ds(2 * HALF + k * 16, 16)],
                         accvm[pl.ds(3 * HALF + k * 16, 16)]]

            def rbody(r, acc):
                out = []
                for k in range(8):
                    xv = rowv[rb + r, pl.ds(k * 16, 16)]
                    s, q, mn, mx = acc[4 * k:4 * k + 4]
                    out += [s + xv, q + xv * xv,
                            jnp.minimum(mn, xv), jnp.maximum(mx, xv)]
                return tuple(out)

            accs = lax.fori_loop(0, rows, rbody, tuple(accs))
            for k in range(8):
                accvm[pl.ds(k * 16, 16)] = accs[4 * k]
                accvm[pl.ds(HALF + k * 16, 16)] = accs[4 * k + 1]
                accvm[pl.ds(2 * HALF + k * 16, 16)] = accs[4 * k + 2]
                accvm[pl.ds(3 * HALF + k * 16, 16)] = accs[4 * k + 3]

            done = run_end == e1

            @pl.when(done)
            def _flush():
                pltpu.sync_copy(accvm, out_hbm.at[h, v])
                _acc_init(accvm)

            v_new = jnp.where(done, walk(v + 1, run_end), v)
            e1_new = jnp.where(done,
                               jnp.where(v_new < v1,
                                         _ext(offv, v_new + 1), e_hi),
                               e1)
            return (run_end, v_new, e1_new)

        return lax.while_loop(lambda c: c[0] < ce, run, carry)

    def chunk_pair(m2, carry):
        for b in range(2):
            m = m2 * 2 + b
            idx_n, row_n, sem_n = (idxB, rowB, semB) if b == 0 else (idxA, rowA, semA)
            idx_c, row_c, sem_c = (idxA, rowA, semA) if b == 0 else (idxB, rowB, semB)

            @pl.when(m + 1 < n_chunks)
            def _start():
                stage_and_start(m + 1, idx_n, row_n, sem_n)

            @pl.when(m < n_chunks)
            def _wait():
                pltpu.make_async_copy(b2_hbm.at[idx_c], row_c, sem_c).wait()

            carry = process(m, carry, row_c)
        return carry

    n2 = (n_chunks + 1) // 2
    lax.fori_loop(0, n2, chunk_pair, (e_lo, v_init, e1_init))


def _sc_segment_reduce(b2, src_s, off, vb):
    mesh = plsc.VectorSubcoreMesh(core_axis_name="c", subcore_axis_name="s",
                                  num_cores=2, num_subcores=16)
    return pl.kernel(
        _sc_body,
        out_type=jax.ShapeDtypeStruct((2, N_NODES, 4 * HALF), jnp.float32),
        mesh=mesh,
        compiler_params=pltpu.CompilerParams(needs_layout_passes=False),
        scratch_types=[
            pltpu.VMEM((OFF_PAD,), jnp.int32),
            pltpu.VMEM((24,), jnp.int32),
            pltpu.VMEM((CHUNK,), jnp.int32),
            pltpu.VMEM((CHUNK,), jnp.int32),
            pltpu.VMEM((CHUNK, HALF), jnp.float32),
            pltpu.VMEM((CHUNK, HALF), jnp.float32),
            pltpu.VMEM((4 * HALF,), jnp.float32),
            pltpu.SemaphoreType.DMA,
            pltpu.SemaphoreType.DMA,
        ],
    )(b2, src_s, off, vb)


def _post_body(x_ref, av_ref, rec_ref, deg_ref,
               wpx_ref, wpid_ref, wpamp_ref, wpatt_ref, bpost_ref,
               wlin_ref, blin_ref, out_ref, *, relu):
    d = deg_ref[:, 0:1]
    cnt = jnp.maximum(d, 1.0)
    has = d > 0.0
    av = av_ref[:]
    h0 = rec_ref[0]
    h1 = rec_ref[1]
    cat = lambda a: jnp.concatenate(
        [h0[:, a * HALF:(a + 1) * HALF], h1[:, a * HALF:(a + 1) * HALF]], axis=1)
    sb = jnp.where(has, cat(0), 0.0)
    sq = jnp.where(has, cat(1), 0.0)
    s = d * av + sb
    mean = s / cnt
    msq = (d * av * av + 2.0 * av * sb + sq) / cnt
    std = jnp.sqrt(jax.nn.relu(msq - mean * mean) + 1e-5)
    mn = jnp.where(has, av + cat(2), 0.0)
    mx = jnp.where(has, av + cat(3), 0.0)
    agg = jnp.concatenate([mean, s, std, mn, mx], axis=1)
    dl = jnp.log(cnt + 1.0)
    a_s = dl / AVG_LOG
    t_s = AVG_LOG / dl
    dot = functools.partial(jnp.dot, preferred_element_type=jnp.float32)
    out = (dot(x_ref[:], wpx_ref[:]) + dot(agg, wpid_ref[:])
           + dot(agg * a_s, wpamp_ref[:]) + dot(agg * t_s, wpatt_ref[:])
           + bpost_ref[:])
    out = dot(out, wlin_ref[:]) + blin_ref[:]
    if relu:
        out = jax.nn.relu(out)
    out_ref[:] = out


def _pna_post(x, av, rec, deg, Wpost, bpost, Wlin, blin, relu):
    wpx = Wpost[0:DIM]
    wpid = Wpost[DIM:DIM + 5 * DIM]
    wpamp = Wpost[DIM + 5 * DIM:DIM + 10 * DIM]
    wpatt = Wpost[DIM + 10 * DIM:DIM + 15 * DIM]
    n_blk = N_NODES // BLK
    row = lambda i: (i, 0)
    full = lambda i: (0, 0)
    grid_spec = pl.GridSpec(
        grid=(n_blk,),
        in_specs=[
            pl.BlockSpec((BLK, DIM), row),
            pl.BlockSpec((BLK, DIM), row),
            pl.BlockSpec((2, BLK, 4 * HALF), lambda i: (0, i, 0)),
            pl.BlockSpec((BLK, 1), row),
            pl.BlockSpec((DIM, DIM), full),
            pl.BlockSpec((5 * DIM, DIM), full),
            pl.BlockSpec((5 * DIM, DIM), full),
            pl.BlockSpec((5 * DIM, DIM), full),
            pl.BlockSpec((1, DIM), full),
            pl.BlockSpec((DIM, DIM), full),
            pl.BlockSpec((1, DIM), full),
        ],
        out_specs=pl.BlockSpec((BLK, DIM), row),
    )
    return pl.pallas_call(
        functools.partial(_post_body, relu=relu),
        grid_spec=grid_spec,
        out_shape=jax.ShapeDtypeStruct((N_NODES, DIM), jnp.float32),
    )(x, av, rec, deg, wpx, wpid, wpamp, wpatt,
      bpost[None, :], Wlin, blin[None, :])


def _layer(x, src_s, off, vb, deg, Wpre, bpre, Wpost, bpost, Wlin, blin, relu):
    a = x @ Wpre[:DIM] + bpre
    b = x @ Wpre[DIM:]
    b2 = b.reshape(2 * N_NODES, HALF)
    rec = _sc_segment_reduce(b2, src_s, off, vb)
    return _pna_post(x, a, rec, deg[:, None], Wpost, bpost, Wlin, blin, relu)


def kernel(x, edge_index, Wpre0, bpre0, Wpost0, bpost0, Wlin0, blin0,
           Wpre1, bpre1, Wpost1, bpost1, Wlin1, blin1, Wout, bout):
    src = edge_index[0]
    dst = edge_index[1]
    order = jnp.argsort(dst)
    src_s = jnp.take(src, order).astype(jnp.int32)
    dst_s = jnp.take(dst, order).astype(jnp.int32)
    off = jnp.searchsorted(dst_s, jnp.arange(N_NODES + 1, dtype=jnp.int32),
                           side="left").astype(jnp.int32)
    deg = (off[1:] - off[:-1]).astype(jnp.float32)
    targets = (jnp.arange(NPART + 1, dtype=jnp.int32) * (N_EDGES // NPART))
    vb = jnp.searchsorted(off, targets, side="left").astype(jnp.int32)
    vb = vb.at[0].set(0).at[NPART].set(N_NODES)
    src_sp = jnp.pad(src_s, (0, SP - N_EDGES))
    off_p = jnp.pad(off, (0, OFF_PAD - (N_NODES + 1)),
                    constant_values=N_EDGES)
    vb_p = jnp.pad(vb, (0, 24 - (NPART + 1)))

    h = _layer(x, src_sp, off_p, vb_p, deg, Wpre0, bpre0, Wpost0, bpost0,
               Wlin0, blin0, relu=True)
    h = _layer(h, src_sp, off_p, vb_p, deg, Wpre1, bpre1, Wpost1, bpost1,
               Wlin1, blin1, relu=True)
    return jnp.squeeze(h @ Wout + bout)
